# baseline (speedup 1.0000x reference)
import jax
import jax.numpy as jnp
from jax import lax
from jax.experimental import pallas as pl
from jax.experimental.pallas import tpu as pltpu
from jax.experimental.pallas import tpu_sc as plsc

_DIM = 4096
_E = 64
_K = 8
_SCALE = 2.5
_TOK = 16384
_NW = 32
_L = 16
_C = 8
_CTOK = _TOK // _C


def _logits_body(x_ref, wt_ref, o_ref):
    acc = jax.lax.dot_general(
        x_ref[...], wt_ref[...], (((1,), (0,)), ((), ())),
        preferred_element_type=jnp.float32)
    o_ref[...] = acc * _SCALE


def _logits(x, wt, bt):
    ntok = x.shape[0]
    return pl.pallas_call(
        _logits_body,
        grid=(ntok // bt,),
        in_specs=[
            pl.BlockSpec((bt, _DIM), lambda i: (i, 0)),
            pl.BlockSpec((_DIM, _E), lambda i: (0, 0)),
        ],
        out_specs=pl.BlockSpec((bt, _E), lambda i: (i, 0)),
        out_shape=jax.ShapeDtypeStruct((ntok, _E), jnp.float32),
        compiler_params=pltpu.CompilerParams(
            dimension_semantics=("arbitrary",)),
    )(x, wt)


def _route_body(tpw, l_hbm, w_hbm, i_hbm, f_hbm, p_hbm,
                l_vm, w_vm, i_vm, f_vm, p_vm):
    cid = lax.axis_index("c")
    sid = lax.axis_index("s")
    wid = sid * 2 + cid
    base = wid * tpw
    pltpu.sync_copy(l_hbm.at[pl.ds(base, tpw), :], l_vm)

    lane = lax.iota(jnp.int32, _L)
    m8 = lane < _K
    idx = [lane + j * _L for j in range(4)]
    ones = jnp.ones((_L,), jnp.float32)
    zeros = jnp.zeros((_L,), jnp.float32)
    for j in range(4):
        f_vm[pl.ds(j * _L, _L)] = zeros

    def merge(ka, va, kb, vb):
        ck = jnp.where(m8, ka, jnp.flip(kb))
        cv = jnp.where(m8, va, jnp.flip(vb))
        return plsc.sort_key_val(ck, cv, descending=True)

    def body(t, p_acc):
        s = [l_vm[t, pl.ds(j * _L, _L)] for j in range(4)]
        mx = jnp.max(jnp.maximum(jnp.maximum(s[0], s[1]),
                                 jnp.maximum(s[2], s[3])))
        e = [jnp.exp(sj - mx) for sj in s]
        tot = jnp.sum(e[0] + e[1] + e[2] + e[3])
        r = ones / tot
        p_acc = tuple(p_acc[j] + e[j] * r for j in range(4))
        kv = [plsc.sort_key_val(e[j], idx[j], descending=True)
              for j in range(4)]
        ka, va = merge(kv[0][0], kv[0][1], kv[1][0], kv[1][1])
        kb, vb = merge(kv[2][0], kv[2][1], kv[3][0], kv[3][1])
        kt, vt = merge(ka, va, kb, vb)
        s8 = jnp.sum(jnp.where(m8, kt, 0.0))
        wv = kt / s8
        w_vm[pl.ds(t * _K, _L)] = wv
        i_vm[pl.ds(t * _K, _L)] = vt
        plsc.addupdate_scatter(f_vm, [vt], ones, mask=m8)
        return p_acc

    p_acc = lax.fori_loop(0, tpw, body, (zeros, zeros, zeros, zeros))
    for j in range(4):
        p_vm[pl.ds(j * _L, _L)] = p_acc[j]

    n = tpw * _K
    pltpu.sync_copy(w_vm.at[pl.ds(0, n)], w_hbm.at[pl.ds(base * _K, n)])
    pltpu.sync_copy(i_vm.at[pl.ds(0, n)], i_hbm.at[pl.ds(base * _K, n)])
    pltpu.sync_copy(f_vm, f_hbm.at[wid])
    pltpu.sync_copy(p_vm, p_hbm.at[wid])


def _route(logits):
    ntok = logits.shape[0]
    tpw = ntok // _NW
    mesh = plsc.VectorSubcoreMesh(core_axis_name="c", subcore_axis_name="s")
    return pl.kernel(
        lambda *refs: _route_body(tpw, *refs),
        out_type=(
            jax.ShapeDtypeStruct((ntok * _K,), jnp.float32),
            jax.ShapeDtypeStruct((ntok * _K,), jnp.int32),
            jax.ShapeDtypeStruct((_NW, _E), jnp.float32),
            jax.ShapeDtypeStruct((_NW, _E), jnp.float32),
        ),
        mesh=mesh,
        scratch_types=[
            pltpu.VMEM((tpw, _E), jnp.float32),
            pltpu.VMEM((tpw * _K + _L,), jnp.float32),
            pltpu.VMEM((tpw * _K + _L,), jnp.int32),
            pltpu.VMEM((_E,), jnp.float32),
            pltpu.VMEM((_E,), jnp.float32),
        ],
        compiler_params=pltpu.CompilerParams(needs_layout_passes=False),
    )(logits)


def _aux_body(f_ref, p_ref, o_ref):
    fsum = jnp.sum(f_ref[...], axis=0)
    psum = jnp.sum(p_ref[...], axis=0)
    o_ref[0, 0] = jnp.sum(fsum * psum) * (_E / (_TOK * _TOK))


def _aux(f_part, p_part):
    return pl.pallas_call(
        _aux_body,
        out_specs=pl.BlockSpec(memory_space=pltpu.SMEM),
        out_shape=jax.ShapeDtypeStruct((1, 1), jnp.float32),
    )(f_part, p_part)


def kernel(x, W):
    wt = W.T
    ws, is_, fs, ps = [], [], [], []
    for c in range(_C):
        logits = _logits(lax.slice_in_dim(x, c * _CTOK, (c + 1) * _CTOK), wt,
                         bt=min(_CTOK, 1024))
        w_c, i_c, f_c, p_c = _route(logits)
        ws.append(w_c)
        is_.append(i_c)
        fs.append(f_c)
        ps.append(p_c)
    aux = _aux(jnp.concatenate(fs, axis=0), jnp.concatenate(ps, axis=0))
    w_flat = jnp.concatenate(ws)
    i_flat = jnp.concatenate(is_)
    return (w_flat.reshape(_TOK, _K),
            i_flat.reshape(_TOK, _K),
            aux[0, 0])


# speedup vs baseline: 2.1462x; 2.1462x over previous
import jax
import jax.numpy as jnp
from jax import lax
from jax.experimental import pallas as pl
from jax.experimental.pallas import tpu as pltpu
from jax.experimental.pallas import tpu_sc as plsc

_DIM = 4096
_E = 64
_K = 8
_SCALE = 2.5
_TOK = 16384
_NW = 32
_TPW = _TOK // _NW
_L = 16


def _logits_body(x_ref, wt_ref, e_ref, ps_ref):
    i = pl.program_id(0)
    logits = jax.lax.dot_general(
        x_ref[...], wt_ref[...], (((1,), (0,)), ((), ())),
        preferred_element_type=jnp.float32) * _SCALE
    mx = jnp.max(logits, axis=1, keepdims=True)
    ee = jnp.exp(logits - mx)
    e_ref[...] = ee
    tot = jnp.sum(ee, axis=1, keepdims=True)
    pp = jnp.sum(ee / tot, axis=0, keepdims=True)

    @pl.when(i == 0)
    def _init():
        ps_ref[...] = jnp.zeros_like(ps_ref)

    ps_ref[...] += pp


def _logits(x, wt, bt=1024):
    return pl.pallas_call(
        _logits_body,
        grid=(_TOK // bt,),
        in_specs=[
            pl.BlockSpec((bt, _DIM), lambda i: (i, 0)),
            pl.BlockSpec((_DIM, _E), lambda i: (0, 0)),
        ],
        out_specs=[
            pl.BlockSpec((bt, _E), lambda i: (i, 0)),
            pl.BlockSpec((1, _E), lambda i: (0, 0)),
        ],
        out_shape=[
            jax.ShapeDtypeStruct((_TOK, _E), jnp.float32),
            jax.ShapeDtypeStruct((1, _E), jnp.float32),
        ],
        compiler_params=pltpu.CompilerParams(
            dimension_semantics=("arbitrary",)),
    )(x, wt)


def _route_body(l_hbm, w_hbm, i_hbm, f_hbm, l_vm, w_vm, i_vm, f_vm):
    cid = lax.axis_index("c")
    sid = lax.axis_index("s")
    wid = sid * 2 + cid
    base = wid * _TPW
    pltpu.sync_copy(l_hbm.at[pl.ds(base, _TPW), :], l_vm)

    lane = lax.iota(jnp.int32, _L)
    m8 = lane < _K
    idx = [lane + j * _L for j in range(4)]
    ones = jnp.ones((_L,), jnp.float32)
    zeros = jnp.zeros((_L,), jnp.float32)
    for j in range(4):
        f_vm[pl.ds(j * _L, _L)] = zeros

    def merge(ka, va, kb, vb):
        ck = jnp.where(m8, ka, jnp.flip(kb))
        cv = jnp.where(m8, va, jnp.flip(vb))
        return plsc.sort_key_val(ck, cv, descending=True)

    def body(t, carry):
        s = [l_vm[t, pl.ds(j * _L, _L)] for j in range(4)]
        kv = [plsc.sort_key_val(s[j], idx[j], descending=True)
              for j in range(4)]
        ka, va = merge(kv[0][0], kv[0][1], kv[1][0], kv[1][1])
        kb, vb = merge(kv[2][0], kv[2][1], kv[3][0], kv[3][1])
        kt, vt = merge(ka, va, kb, vb)
        s8 = jnp.sum(jnp.where(m8, kt, 0.0))
        wv = kt / s8
        w_vm[pl.ds(t * _K, _L)] = wv
        i_vm[pl.ds(t * _K, _L)] = vt
        plsc.addupdate_scatter(f_vm, [vt], ones, mask=m8)
        return carry

    lax.fori_loop(0, _TPW, body, 0, unroll=4)

    n = _TPW * _K
    pltpu.sync_copy(w_vm.at[pl.ds(0, n)], w_hbm.at[pl.ds(base * _K, n)])
    pltpu.sync_copy(i_vm.at[pl.ds(0, n)], i_hbm.at[pl.ds(base * _K, n)])
    pltpu.sync_copy(f_vm, f_hbm.at[wid])


def _route(evals):
    mesh = plsc.VectorSubcoreMesh(core_axis_name="c", subcore_axis_name="s")
    return pl.kernel(
        _route_body,
        out_type=(
            jax.ShapeDtypeStruct((_TOK * _K,), jnp.float32),
            jax.ShapeDtypeStruct((_TOK * _K,), jnp.int32),
            jax.ShapeDtypeStruct((_NW, _E), jnp.float32),
        ),
        mesh=mesh,
        scratch_types=[
            pltpu.VMEM((_TPW, _E), jnp.float32),
            pltpu.VMEM((_TPW * _K + _L,), jnp.float32),
            pltpu.VMEM((_TPW * _K + _L,), jnp.int32),
            pltpu.VMEM((_E,), jnp.float32),
        ],
        compiler_params=pltpu.CompilerParams(needs_layout_passes=False),
    )(evals)


def _aux_body(f_ref, p_ref, o_ref):
    fsum = jnp.sum(f_ref[...], axis=0)
    o_ref[0, 0] = jnp.sum(fsum * p_ref[0, :]) * (_E / (_TOK * _TOK))


def _aux(f_part, p_sum):
    return pl.pallas_call(
        _aux_body,
        out_specs=pl.BlockSpec(memory_space=pltpu.SMEM),
        out_shape=jax.ShapeDtypeStruct((1, 1), jnp.float32),
    )(f_part, p_sum)


def kernel(x, W):
    wt = W.T
    evals, p_sum = _logits(x, wt)
    w_flat, i_flat, f_part = _route(evals)
    aux = _aux(f_part, p_sum)
    return (w_flat.reshape(_TOK, _K),
            i_flat.reshape(_TOK, _K),
            aux[0, 0])


# speedup vs baseline: 2.4867x; 1.1587x over previous
import jax
import jax.numpy as jnp
from jax import lax
from jax.experimental import pallas as pl
from jax.experimental.pallas import tpu as pltpu
from jax.experimental.pallas import tpu_sc as plsc

_DIM = 4096
_E = 64
_K = 8
_SCALE = 2.5
_TOK = 16384
_NW = 32
_TPW = _TOK // _NW
_L = 16


def _logits_body(x_ref, wt_ref, e_ref, ps_ref):
    i = pl.program_id(0)
    logits = jax.lax.dot_general(
        x_ref[...], wt_ref[...], (((1,), (0,)), ((), ())),
        preferred_element_type=jnp.float32) * _SCALE
    mx = jnp.max(logits, axis=1, keepdims=True)
    ee = jnp.exp(logits - mx)
    e_ref[...] = ee
    tot = jnp.sum(ee, axis=1, keepdims=True)
    pp = jnp.sum(ee / tot, axis=0, keepdims=True)

    @pl.when(i == 0)
    def _init():
        ps_ref[...] = jnp.zeros_like(ps_ref)

    ps_ref[...] += pp


def _logits(x, wt, bt=1024):
    return pl.pallas_call(
        _logits_body,
        grid=(_TOK // bt,),
        in_specs=[
            pl.BlockSpec((bt, _DIM), lambda i: (i, 0)),
            pl.BlockSpec((_DIM, _E), lambda i: (0, 0)),
        ],
        out_specs=[
            pl.BlockSpec((bt, _E), lambda i: (i, 0)),
            pl.BlockSpec((1, _E), lambda i: (0, 0)),
        ],
        out_shape=[
            jax.ShapeDtypeStruct((_TOK, _E), jnp.float32),
            jax.ShapeDtypeStruct((1, _E), jnp.float32),
        ],
        compiler_params=pltpu.CompilerParams(
            dimension_semantics=("arbitrary",)),
    )(x, wt)


def _route_body(l_hbm, w_hbm, i_hbm, f_hbm, l_vm, w_vm, i_vm, f_vm):
    cid = lax.axis_index("c")
    sid = lax.axis_index("s")
    wid = sid * 2 + cid
    base = wid * _TPW
    pltpu.sync_copy(l_hbm.at[pl.ds(base, _TPW), :], l_vm)

    lane = lax.iota(jnp.int32, _L)
    m8 = lane < _K
    idx = [lane + j * _L for j in range(4)]
    ones = jnp.ones((_L,), jnp.float32)
    zeros = jnp.zeros((_L,), jnp.float32)
    for j in range(4):
        f_vm[pl.ds(j * _L, _L)] = zeros

    def merge(ka, va, kb, vb):
        ck = jnp.where(m8, ka, jnp.flip(kb))
        cv = jnp.where(m8, va, jnp.flip(vb))
        return plsc.sort_key_val(ck, cv, descending=True)

    @plsc.parallel_loop(0, _TPW, 1, unroll=4)
    def _token(t):
        s = [l_vm[t, pl.ds(j * _L, _L)] for j in range(4)]
        kv = [plsc.sort_key_val(s[j], idx[j], descending=True)
              for j in range(4)]
        ka, va = merge(kv[0][0], kv[0][1], kv[1][0], kv[1][1])
        kb, vb = merge(kv[2][0], kv[2][1], kv[3][0], kv[3][1])
        kt, vt = merge(ka, va, kb, vb)
        s8 = jnp.sum(jnp.where(m8, kt, 0.0))
        wv = kt / s8
        w_vm[pl.ds(t * _K, _L)] = wv
        i_vm[pl.ds(t * _K, _L)] = vt
        plsc.addupdate_scatter(f_vm, [vt], ones, mask=m8)

    n = _TPW * _K
    pltpu.sync_copy(w_vm.at[pl.ds(0, n)], w_hbm.at[pl.ds(base * _K, n)])
    pltpu.sync_copy(i_vm.at[pl.ds(0, n)], i_hbm.at[pl.ds(base * _K, n)])
    pltpu.sync_copy(f_vm, f_hbm.at[wid])


def _route(evals):
    mesh = plsc.VectorSubcoreMesh(core_axis_name="c", subcore_axis_name="s")
    return pl.kernel(
        _route_body,
        out_type=(
            jax.ShapeDtypeStruct((_TOK * _K,), jnp.float32),
            jax.ShapeDtypeStruct((_TOK * _K,), jnp.int32),
            jax.ShapeDtypeStruct((_NW, _E), jnp.float32),
        ),
        mesh=mesh,
        scratch_types=[
            pltpu.VMEM((_TPW, _E), jnp.float32),
            pltpu.VMEM((_TPW * _K + _L,), jnp.float32),
            pltpu.VMEM((_TPW * _K + _L,), jnp.int32),
            pltpu.VMEM((_E,), jnp.float32),
        ],
        compiler_params=pltpu.CompilerParams(needs_layout_passes=False),
    )(evals)


def _aux_body(f_ref, p_ref, o_ref):
    fsum = jnp.sum(f_ref[...], axis=0)
    o_ref[0, 0] = jnp.sum(fsum * p_ref[0, :]) * (_E / (_TOK * _TOK))


def _aux(f_part, p_sum):
    return pl.pallas_call(
        _aux_body,
        out_specs=pl.BlockSpec(memory_space=pltpu.SMEM),
        out_shape=jax.ShapeDtypeStruct((1, 1), jnp.float32),
    )(f_part, p_sum)


def kernel(x, W):
    wt = W.T
    evals, p_sum = _logits(x, wt)
    w_flat, i_flat, f_part = _route(evals)
    aux = _aux(f_part, p_sum)
    return (w_flat.reshape(_TOK, _K),
            i_flat.reshape(_TOK, _K),
            aux[0, 0])
